# Initial kernel scaffold; baseline (speedup 1.0000x reference)
import jax
import jax.numpy as jnp
from jax.experimental import pallas as pl


def kernel(z, dictionary):
    raise NotImplementedError("write your pallas kernel here")


# speedup vs baseline: 20.4543x; 20.4543x over previous
import jax
import jax.numpy as jnp
from jax.experimental import pallas as pl
from jax.experimental.pallas import tpu as pltpu

_NUM_EMBEDDINGS = 1024
_EMBEDDING_DIM = 16
_PATCH_SIZE = 8
_PATCH_STRIDE = 4
_SPARSITY = 8
_DIAG_EPS = 1e-4
_CHOL_EPS = 1e-6
_PATCH_DIM = _PATCH_SIZE * _PATCH_SIZE * _EMBEDDING_DIM

_BLOCK_P = 256


def _gram_body(d_ref, g_ref):
    d = d_ref[...]
    k = d.shape[1]
    g = jax.lax.dot_general(d, d, (((0,), (0,)), ((), ())),
                            preferred_element_type=jnp.float32)
    row = jax.lax.broadcasted_iota(jnp.int32, (k, k), 0)
    col = jax.lax.broadcasted_iota(jnp.int32, (k, k), 1)
    g_ref[...] = g + jnp.where(row == col, jnp.float32(_DIAG_EPS),
                               jnp.float32(0.0))


def _chol_solve(A, b, m):
    L = [[None] * m for _ in range(m)]
    Linv = [None] * m
    for i in range(m):
        for j in range(i):
            s = A[i][j]
            for k2 in range(j):
                s = s - L[i][k2] * L[j][k2]
            L[i][j] = s * Linv[j]
        s = A[i][i] + jnp.float32(_CHOL_EPS)
        for k2 in range(i):
            s = s - L[i][k2] * L[i][k2]
        diag = jnp.sqrt(jnp.maximum(s, jnp.float32(1e-20)))
        L[i][i] = diag
        Linv[i] = jnp.float32(1.0) / diag
    y = []
    for i in range(m):
        s = b[i]
        for k2 in range(i):
            s = s - L[i][k2] * y[k2]
        y.append(s * Linv[i])
    x = [None] * m
    for i in reversed(range(m)):
        s = y[i]
        for k2 in range(i + 1, m):
            s = s - L[k2][i] * x[k2]
        x[i] = s * Linv[i]
    return x


def _omp_body(x_ref, d_ref, g_ref, out_ref):
    P = x_ref.shape[0]
    K = _NUM_EMBEDDINGS
    x = x_ref[...]
    d = d_ref[...]
    g = g_ref[...]

    h_bar = jax.lax.dot_general(x, d, (((1,), (0,)), ((), ())),
                                preferred_element_type=jnp.float32)
    h = h_bar
    masked = jnp.zeros((P, K), dtype=jnp.bool_)
    iota_k = jax.lax.broadcasted_iota(jnp.int32, (P, K), 1)

    idxs = []
    Gsm = [[None] * _SPARSITY for _ in range(_SPARSITY)]
    h_I = []
    w = None

    for t in range(_SPARSITY):
        scores = jnp.where(masked, jnp.float32(-1.0), jnp.abs(h))
        smax = jnp.max(scores, axis=1, keepdims=True)
        is_max = scores == smax
        idx = jnp.min(jnp.where(is_max, iota_k, jnp.int32(K)),
                      axis=1, keepdims=True)
        onehot_b = iota_k == idx
        onehot = onehot_b.astype(jnp.float32)
        masked = jnp.logical_or(masked, onehot_b)
        idxs.append(idx)

        grow = jax.lax.dot_general(onehot, g, (((1,), (0,)), ((), ())),
                                   preferred_element_type=jnp.float32,
                                   precision=jax.lax.Precision.HIGHEST)

        for a in range(t):
            oh_a = (iota_k == idxs[a]).astype(jnp.float32)
            v = jnp.sum(grow * oh_a, axis=1, keepdims=True)
            Gsm[a][t] = v
            Gsm[t][a] = v
        Gsm[t][t] = jnp.sum(grow * onehot, axis=1, keepdims=True)
        h_I.append(jnp.sum(h_bar * onehot, axis=1, keepdims=True))

        coeffs = _chol_solve(Gsm, h_I, t + 1)

        w = coeffs[0] * (iota_k == idxs[0]).astype(jnp.float32)
        for a in range(1, t + 1):
            w = w + coeffs[a] * (iota_k == idxs[a]).astype(jnp.float32)
        if t < _SPARSITY - 1:
            beta = jax.lax.dot_general(w, g, (((1,), (0,)), ((), ())),
                                       preferred_element_type=jnp.float32)
            h = h_bar - beta

    out_ref[...] = jax.lax.dot_general(w, d, (((1,), (1,)), ((), ())),
                                       preferred_element_type=jnp.float32)


def _extract(z):
    B, C, H, W = z.shape
    ps, stride = _PATCH_SIZE, _PATCH_STRIDE
    pad = (ps - stride) // 2
    zp = jnp.pad(z, ((0, 0), (0, 0), (pad, pad), (pad, pad)))
    nph = H // stride
    npw = W // stride
    ih = (jnp.arange(nph) * stride)[:, None] + jnp.arange(ps)[None, :]
    iw = (jnp.arange(npw) * stride)[:, None] + jnp.arange(ps)[None, :]
    patches = zp[:, :, ih[:, None, :, None], iw[None, :, None, :]]
    patches = jnp.transpose(patches, (0, 2, 3, 1, 4, 5)).reshape(
        B, nph, npw, C * ps * ps)
    return patches, nph, npw


def kernel(z, dictionary):
    B, C, H, W = z.shape
    patches, nph, npw = _extract(z)
    Xr = patches.reshape(-1, _PATCH_DIM)
    N = Xr.shape[0]
    K = _NUM_EMBEDDINGS

    G = pl.pallas_call(
        _gram_body,
        out_shape=jax.ShapeDtypeStruct((K, K), jnp.float32),
    )(dictionary)

    num_blocks = N // _BLOCK_P
    recon = pl.pallas_call(
        _omp_body,
        grid=(num_blocks,),
        in_specs=[
            pl.BlockSpec((_BLOCK_P, _PATCH_DIM), lambda i: (i, 0)),
            pl.BlockSpec((_PATCH_DIM, K), lambda i: (0, 0)),
            pl.BlockSpec((K, K), lambda i: (0, 0)),
        ],
        out_specs=pl.BlockSpec((_BLOCK_P, _PATCH_DIM), lambda i: (i, 0)),
        out_shape=jax.ShapeDtypeStruct((N, _PATCH_DIM), jnp.float32),
    )(Xr, dictionary, G)

    recon = recon.reshape(B, nph, npw, C, _PATCH_SIZE, _PATCH_SIZE)
    off = (_PATCH_SIZE - _PATCH_STRIDE) // 2
    center = recon[..., off:off + _PATCH_STRIDE, off:off + _PATCH_STRIDE]
    out = jnp.transpose(center, (0, 3, 1, 4, 2, 5)).reshape(B, C, H, W)
    return out


# speedup vs baseline: 26.8645x; 1.3134x over previous
import jax
import jax.numpy as jnp
from jax.experimental import pallas as pl
from jax.experimental.pallas import tpu as pltpu

_NUM_EMBEDDINGS = 1024
_EMBEDDING_DIM = 16
_PATCH_SIZE = 8
_PATCH_STRIDE = 4
_SPARSITY = 8
_DIAG_EPS = 1e-4
_CHOL_EPS = 1e-6
_PATCH_DIM = _PATCH_SIZE * _PATCH_SIZE * _EMBEDDING_DIM

_BLOCK_P = 256


def _gram_body(d_ref, g_ref):
    d = d_ref[...]
    k = d.shape[1]
    g = jax.lax.dot_general(d, d, (((0,), (0,)), ((), ())),
                            preferred_element_type=jnp.float32)
    row = jax.lax.broadcasted_iota(jnp.int32, (k, k), 0)
    col = jax.lax.broadcasted_iota(jnp.int32, (k, k), 1)
    g_ref[...] = g + jnp.where(row == col, jnp.float32(_DIAG_EPS),
                               jnp.float32(0.0))


def _chol_solve(A, b, m):
    L = [[None] * m for _ in range(m)]
    Linv = [None] * m
    for i in range(m):
        for j in range(i):
            s = A[i][j]
            for k2 in range(j):
                s = s - L[i][k2] * L[j][k2]
            L[i][j] = s * Linv[j]
        s = A[i][i] + jnp.float32(_CHOL_EPS)
        for k2 in range(i):
            s = s - L[i][k2] * L[i][k2]
        diag = jnp.sqrt(jnp.maximum(s, jnp.float32(1e-20)))
        L[i][i] = diag
        Linv[i] = jnp.float32(1.0) / diag
    y = []
    for i in range(m):
        s = b[i]
        for k2 in range(i):
            s = s - L[i][k2] * y[k2]
        y.append(s * Linv[i])
    x = [None] * m
    for i in reversed(range(m)):
        s = y[i]
        for k2 in range(i + 1, m):
            s = s - L[k2][i] * x[k2]
        x[i] = s * Linv[i]
    return x


def _omp_body(x_ref, d_ref, g_ref, out_ref):
    P = x_ref.shape[0]
    K = _NUM_EMBEDDINGS
    x = x_ref[...]
    d = d_ref[...]
    g = g_ref[...]

    h_bar = jax.lax.dot_general(x, d, (((1,), (0,)), ((), ())),
                                preferred_element_type=jnp.float32)
    h = h_bar
    masked = jnp.zeros((P, K), dtype=jnp.bool_)
    iota_k = jax.lax.broadcasted_iota(jnp.int32, (P, K), 1)

    g_hi = g.astype(jnp.bfloat16)
    r1 = g - g_hi.astype(jnp.float32)
    g_mid = r1.astype(jnp.bfloat16)
    g_lo = (r1 - g_mid.astype(jnp.float32)).astype(jnp.bfloat16)

    idxs = []
    Gsm = [[None] * _SPARSITY for _ in range(_SPARSITY)]
    h_I = []
    w = None

    for t in range(_SPARSITY):
        scores = jnp.where(masked, jnp.float32(-1.0), jnp.abs(h))
        smax = jnp.max(scores, axis=1, keepdims=True)
        is_max = scores == smax
        idx = jnp.min(jnp.where(is_max, iota_k, jnp.int32(K)),
                      axis=1, keepdims=True)
        onehot_b = iota_k == idx
        onehot = onehot_b.astype(jnp.float32)
        masked = jnp.logical_or(masked, onehot_b)
        idxs.append(idx)

        onehot_b16 = onehot_b.astype(jnp.bfloat16)
        grow = (jax.lax.dot_general(onehot_b16, g_hi, (((1,), (0,)), ((), ())),
                                    preferred_element_type=jnp.float32)
                + jax.lax.dot_general(onehot_b16, g_mid, (((1,), (0,)), ((), ())),
                                      preferred_element_type=jnp.float32)
                + jax.lax.dot_general(onehot_b16, g_lo, (((1,), (0,)), ((), ())),
                                      preferred_element_type=jnp.float32))

        for a in range(t):
            oh_a = (iota_k == idxs[a]).astype(jnp.float32)
            v = jnp.sum(grow * oh_a, axis=1, keepdims=True)
            Gsm[a][t] = v
            Gsm[t][a] = v
        Gsm[t][t] = jnp.sum(grow * onehot, axis=1, keepdims=True)
        h_I.append(jnp.sum(h_bar * onehot, axis=1, keepdims=True))

        coeffs = _chol_solve(Gsm, h_I, t + 1)

        w = coeffs[0] * (iota_k == idxs[0]).astype(jnp.float32)
        for a in range(1, t + 1):
            w = w + coeffs[a] * (iota_k == idxs[a]).astype(jnp.float32)
        if t < _SPARSITY - 1:
            beta = jax.lax.dot_general(w, g, (((1,), (0,)), ((), ())),
                                       preferred_element_type=jnp.float32)
            h = h_bar - beta

    out_ref[...] = jax.lax.dot_general(w, d, (((1,), (1,)), ((), ())),
                                       preferred_element_type=jnp.float32)


def _extract(z):
    B, C, H, W = z.shape
    ps, stride = _PATCH_SIZE, _PATCH_STRIDE
    pad = (ps - stride) // 2
    zp = jnp.pad(z, ((0, 0), (0, 0), (pad, pad), (pad, pad)))
    nph = H // stride
    npw = W // stride
    ih = (jnp.arange(nph) * stride)[:, None] + jnp.arange(ps)[None, :]
    iw = (jnp.arange(npw) * stride)[:, None] + jnp.arange(ps)[None, :]
    patches = zp[:, :, ih[:, None, :, None], iw[None, :, None, :]]
    patches = jnp.transpose(patches, (0, 2, 3, 1, 4, 5)).reshape(
        B, nph, npw, C * ps * ps)
    return patches, nph, npw


def kernel(z, dictionary):
    B, C, H, W = z.shape
    patches, nph, npw = _extract(z)
    Xr = patches.reshape(-1, _PATCH_DIM)
    N = Xr.shape[0]
    K = _NUM_EMBEDDINGS

    G = pl.pallas_call(
        _gram_body,
        out_shape=jax.ShapeDtypeStruct((K, K), jnp.float32),
    )(dictionary)

    num_blocks = N // _BLOCK_P
    recon = pl.pallas_call(
        _omp_body,
        grid=(num_blocks,),
        in_specs=[
            pl.BlockSpec((_BLOCK_P, _PATCH_DIM), lambda i: (i, 0)),
            pl.BlockSpec((_PATCH_DIM, K), lambda i: (0, 0)),
            pl.BlockSpec((K, K), lambda i: (0, 0)),
        ],
        out_specs=pl.BlockSpec((_BLOCK_P, _PATCH_DIM), lambda i: (i, 0)),
        out_shape=jax.ShapeDtypeStruct((N, _PATCH_DIM), jnp.float32),
    )(Xr, dictionary, G)

    recon = recon.reshape(B, nph, npw, C, _PATCH_SIZE, _PATCH_SIZE)
    off = (_PATCH_SIZE - _PATCH_STRIDE) // 2
    center = recon[..., off:off + _PATCH_STRIDE, off:off + _PATCH_STRIDE]
    out = jnp.transpose(center, (0, 3, 1, 4, 2, 5)).reshape(B, C, H, W)
    return out


# speedup vs baseline: 26.8759x; 1.0004x over previous
import jax
import jax.numpy as jnp
from jax.experimental import pallas as pl
from jax.experimental.pallas import tpu as pltpu

_NUM_EMBEDDINGS = 1024
_EMBEDDING_DIM = 16
_PATCH_SIZE = 8
_PATCH_STRIDE = 4
_SPARSITY = 8
_DIAG_EPS = 1e-4
_CHOL_EPS = 1e-6
_PATCH_DIM = _PATCH_SIZE * _PATCH_SIZE * _EMBEDDING_DIM

_BLOCK_P = 256


def _gram_body(d_ref, g_ref, ghi_ref, gmid_ref, glo_ref):
    d = d_ref[...]
    k = d.shape[1]
    g = jax.lax.dot_general(d, d, (((0,), (0,)), ((), ())),
                            preferred_element_type=jnp.float32)
    row = jax.lax.broadcasted_iota(jnp.int32, (k, k), 0)
    col = jax.lax.broadcasted_iota(jnp.int32, (k, k), 1)
    g = g + jnp.where(row == col, jnp.float32(_DIAG_EPS), jnp.float32(0.0))
    g_ref[...] = g
    g_hi = g.astype(jnp.bfloat16)
    r1 = g - g_hi.astype(jnp.float32)
    g_mid = r1.astype(jnp.bfloat16)
    ghi_ref[...] = g_hi
    gmid_ref[...] = g_mid
    glo_ref[...] = (r1 - g_mid.astype(jnp.float32)).astype(jnp.bfloat16)


def _chol_solve(A, b, m):
    L = [[None] * m for _ in range(m)]
    Linv = [None] * m
    for i in range(m):
        for j in range(i):
            s = A[i][j]
            for k2 in range(j):
                s = s - L[i][k2] * L[j][k2]
            L[i][j] = s * Linv[j]
        s = A[i][i] + jnp.float32(_CHOL_EPS)
        for k2 in range(i):
            s = s - L[i][k2] * L[i][k2]
        diag = jnp.sqrt(jnp.maximum(s, jnp.float32(1e-20)))
        L[i][i] = diag
        Linv[i] = jnp.float32(1.0) / diag
    y = []
    for i in range(m):
        s = b[i]
        for k2 in range(i):
            s = s - L[i][k2] * y[k2]
        y.append(s * Linv[i])
    x = [None] * m
    for i in reversed(range(m)):
        s = y[i]
        for k2 in range(i + 1, m):
            s = s - L[k2][i] * x[k2]
        x[i] = s * Linv[i]
    return x


def _omp_body(x_ref, d_ref, g_ref, ghi_ref, gmid_ref, glo_ref, out_ref):
    P = x_ref.shape[0]
    K = _NUM_EMBEDDINGS
    x = x_ref[...]
    d = d_ref[...]
    g = g_ref[...]
    g_hi = ghi_ref[...]
    g_mid = gmid_ref[...]
    g_lo = glo_ref[...]

    h_bar = jax.lax.dot_general(x, d, (((1,), (0,)), ((), ())),
                                preferred_element_type=jnp.float32)
    h = h_bar
    masked = jnp.zeros((P, K), dtype=jnp.bool_)
    iota_k = jax.lax.broadcasted_iota(jnp.int32, (P, K), 1)

    idxs = []
    Gsm = [[None] * _SPARSITY for _ in range(_SPARSITY)]
    h_I = []
    w = None

    for t in range(_SPARSITY):
        scores = jnp.where(masked, jnp.float32(-1.0), jnp.abs(h))
        smax = jnp.max(scores, axis=1, keepdims=True)
        is_max = scores == smax
        idx = jnp.min(jnp.where(is_max, iota_k, jnp.int32(K)),
                      axis=1, keepdims=True)
        onehot_b = iota_k == idx
        onehot = onehot_b.astype(jnp.float32)
        masked = jnp.logical_or(masked, onehot_b)
        idxs.append(idx)

        onehot_b16 = onehot_b.astype(jnp.bfloat16)
        grow = (jax.lax.dot_general(onehot_b16, g_hi, (((1,), (0,)), ((), ())),
                                    preferred_element_type=jnp.float32)
                + jax.lax.dot_general(onehot_b16, g_mid, (((1,), (0,)), ((), ())),
                                      preferred_element_type=jnp.float32)
                + jax.lax.dot_general(onehot_b16, g_lo, (((1,), (0,)), ((), ())),
                                      preferred_element_type=jnp.float32))

        for a in range(t):
            oh_a = (iota_k == idxs[a]).astype(jnp.float32)
            v = jnp.sum(grow * oh_a, axis=1, keepdims=True)
            Gsm[a][t] = v
            Gsm[t][a] = v
        Gsm[t][t] = jnp.sum(grow * onehot, axis=1, keepdims=True)
        h_I.append(jnp.sum(h_bar * onehot, axis=1, keepdims=True))

        coeffs = _chol_solve(Gsm, h_I, t + 1)

        w = coeffs[0] * (iota_k == idxs[0]).astype(jnp.float32)
        for a in range(1, t + 1):
            w = w + coeffs[a] * (iota_k == idxs[a]).astype(jnp.float32)
        if t < _SPARSITY - 1:
            beta = jax.lax.dot_general(w, g, (((1,), (0,)), ((), ())),
                                       preferred_element_type=jnp.float32)
            h = h_bar - beta

    out_ref[...] = jax.lax.dot_general(w, d, (((1,), (1,)), ((), ())),
                                       preferred_element_type=jnp.float32)


def _extract(z):
    B, C, H, W = z.shape
    ps, stride = _PATCH_SIZE, _PATCH_STRIDE
    pad = (ps - stride) // 2
    zp = jnp.pad(z, ((0, 0), (0, 0), (pad, pad), (pad, pad)))
    nph = H // stride
    npw = W // stride
    ih = (jnp.arange(nph) * stride)[:, None] + jnp.arange(ps)[None, :]
    iw = (jnp.arange(npw) * stride)[:, None] + jnp.arange(ps)[None, :]
    patches = zp[:, :, ih[:, None, :, None], iw[None, :, None, :]]
    patches = jnp.transpose(patches, (0, 2, 3, 1, 4, 5)).reshape(
        B, nph, npw, C * ps * ps)
    return patches, nph, npw


def kernel(z, dictionary):
    B, C, H, W = z.shape
    patches, nph, npw = _extract(z)
    Xr = patches.reshape(-1, _PATCH_DIM)
    N = Xr.shape[0]
    K = _NUM_EMBEDDINGS

    G, G_hi, G_mid, G_lo = pl.pallas_call(
        _gram_body,
        out_shape=(
            jax.ShapeDtypeStruct((K, K), jnp.float32),
            jax.ShapeDtypeStruct((K, K), jnp.bfloat16),
            jax.ShapeDtypeStruct((K, K), jnp.bfloat16),
            jax.ShapeDtypeStruct((K, K), jnp.bfloat16),
        ),
    )(dictionary)

    num_blocks = N // _BLOCK_P
    recon = pl.pallas_call(
        _omp_body,
        grid=(num_blocks,),
        in_specs=[
            pl.BlockSpec((_BLOCK_P, _PATCH_DIM), lambda i: (i, 0)),
            pl.BlockSpec((_PATCH_DIM, K), lambda i: (0, 0)),
            pl.BlockSpec((K, K), lambda i: (0, 0)),
            pl.BlockSpec((K, K), lambda i: (0, 0)),
            pl.BlockSpec((K, K), lambda i: (0, 0)),
            pl.BlockSpec((K, K), lambda i: (0, 0)),
        ],
        out_specs=pl.BlockSpec((_BLOCK_P, _PATCH_DIM), lambda i: (i, 0)),
        out_shape=jax.ShapeDtypeStruct((N, _PATCH_DIM), jnp.float32),
    )(Xr, dictionary, G, G_hi, G_mid, G_lo)

    recon = recon.reshape(B, nph, npw, C, _PATCH_SIZE, _PATCH_SIZE)
    off = (_PATCH_SIZE - _PATCH_STRIDE) // 2
    center = recon[..., off:off + _PATCH_STRIDE, off:off + _PATCH_STRIDE]
    out = jnp.transpose(center, (0, 3, 1, 4, 2, 5)).reshape(B, C, H, W)
    return out
